# Initial kernel scaffold; baseline (speedup 1.0000x reference)
import jax
import jax.numpy as jnp
from jax.experimental import pallas as pl


def kernel(x1, x2, edges, W1, b1, W2, b2, W3, b3, D1W, D1b, D2W, D2b):
    raise NotImplementedError("write your pallas kernel here")


# speedup vs baseline: 49.9110x; 49.9110x over previous
import functools

import jax
import jax.numpy as jnp
from jax import lax
from jax.experimental import pallas as pl
from jax.experimental.pallas import tpu as pltpu
from jax.experimental.pallas import tpu_sc as plsc

NC = 2
NS = 16
NW = NC * NS
LANES = 128
K = 8


def _make_edge_pass(npad, p, sct, t_rows):
    rt = npad // NS
    groups = t_rows // K
    mesh = plsc.VectorSubcoreMesh(core_axis_name="c", subcore_axis_name="s")

    @functools.partial(
        pl.kernel,
        out_type=jax.ShapeDtypeStruct((NC, npad, p), jnp.float32),
        mesh=mesh,
        compiler_params=pltpu.CompilerParams(use_tc_tiling_on_sc=False),
        scratch_types=dict(
            acc=pltpu.VMEM_SHARED((npad, p), jnp.float32),
            sbuf=pltpu.VMEM((K, LANES), jnp.int32),
            dbuf=pltpu.VMEM((K, LANES), jnp.int32),
            rows=pltpu.VMEM((K * LANES, p), jnp.float32),
            gsem=pltpu.SemaphoreType.DMA,
            ssem=pltpu.SemaphoreType.DMA,
        ),
    )
    def edge_pass(table, srcr, dstr, zrows, out, acc, sbuf, dbuf, rows,
                  gsem, ssem):
        c = lax.axis_index("c")
        s = lax.axis_index("s")
        wid = s * NC + c
        rt0 = s * rt
        pltpu.sync_copy(zrows.at[pl.ds(rt0, rt)], acc.at[pl.ds(rt0, rt)])
        plsc.subcore_barrier()

        def group(g, carry):
            base = wid * t_rows + g * K
            pltpu.sync_copy(srcr.at[pl.ds(base, K)], sbuf)
            pltpu.sync_copy(dstr.at[pl.ds(base, K)], dbuf)
            gathers = [
                pltpu.async_copy(table.at[sbuf.at[j]],
                                 rows.at[pl.ds(j * LANES, LANES)], gsem)
                for j in range(K)
            ]
            for gth in gathers:
                gth.wait()
            scatters = [
                pltpu.async_copy(rows.at[pl.ds(j * LANES, LANES)],
                                 acc.at[dbuf.at[j]], ssem, add=True)
                for j in range(K)
            ]
            for sc in scatters:
                sc.wait()
            return carry

        lax.fori_loop(0, groups, group, 0)
        plsc.subcore_barrier()
        pltpu.sync_copy(acc.at[pl.ds(rt0, rt)], out.at[c, pl.ds(rt0, rt)])

    return edge_pass


def kernel(x1, x2, edges, W1, b1, W2, b2, W3, b3, D1W, D1b, D2W, D2b):
    n = x1.shape[0]
    e = edges.shape[1]
    p = 8
    npad = n + 96
    ec = e // LANES
    t_rows = -(-ec // (NW * K)) * K
    sct = NW * t_rows
    pad_e = sct * LANES - e

    pad_idx = n + (jnp.arange(pad_e, dtype=jnp.int32) % 32)
    srcr = jnp.concatenate([edges[0], pad_idx]).reshape(sct, LANES)
    dstr = jnp.concatenate([edges[1], pad_idx]).reshape(sct, LANES)
    zrows = jnp.zeros((npad, p), jnp.float32)

    edge_pass = _make_edge_pass(npad, p, sct, t_rows)

    def pad_table(u):
        return jnp.zeros((npad, p), jnp.float32).at[:n, :u.shape[1]].set(u)

    ones_t = zrows.at[:, 0].set(1.0)
    cnt = edge_pass(ones_t, dstr, dstr, zrows)
    deg = 1.0 + cnt[0, :n, 0] + cnt[1, :n, 0]
    dis = lax.rsqrt(deg)[:, None]

    u1 = dis * (x1 @ W1)
    s1 = edge_pass(pad_table(u1), srcr, dstr, zrows)
    h1 = jax.nn.relu(dis * (s1[0, :n, :5] + s1[1, :n, :5] + u1) + b1)
    u2 = dis * (h1 @ W2)
    s2 = edge_pass(pad_table(u2), srcr, dstr, zrows)
    h2 = jax.nn.relu(dis * (s2[0, :n, :5] + s2[1, :n, :5] + u2) + b2)
    u3 = dis * (h2 @ W3)
    s3 = edge_pass(pad_table(u3), srcr, dstr, zrows)
    h3 = dis * (s3[0, :n, :1] + s3[1, :n, :1] + u3) + b3

    m = h3.mean()
    v = jnp.concatenate([m[None], x2])
    v = jax.nn.relu(v @ D1W + D1b)
    out = (v @ D2W + D2b)[0]
    return jnp.tanh(out)


# speedup vs baseline: 53.0731x; 1.0634x over previous
import functools

import jax
import jax.numpy as jnp
from jax import lax
from jax.experimental import pallas as pl
from jax.experimental.pallas import tpu as pltpu
from jax.experimental.pallas import tpu_sc as plsc

NC = 2
NS = 16
NW = NC * NS
G = 1024
NB = 5


def _make_edge_pass(npad, p, t_edges):
    rt = npad // NS
    t_total = t_edges // (G * NB)
    assert t_total >= 4 and t_total % 2 == 0
    mesh = plsc.VectorSubcoreMesh(core_axis_name="c", subcore_axis_name="s")

    scratch = dict(acc=pltpu.VMEM_SHARED((npad, p), jnp.float32))
    for b in range(NB):
        scratch[f"sidx{b}"] = pltpu.VMEM((2, G), jnp.int32)
        scratch[f"didx{b}"] = pltpu.VMEM((2, G), jnp.int32)
        scratch[f"rows{b}"] = pltpu.VMEM((G, p), jnp.float32)
        scratch[f"gsem{b}"] = pltpu.SemaphoreType.DMA
        scratch[f"ssem{b}"] = pltpu.SemaphoreType.DMA
        for q in range(2):
            scratch[f"isem{b}_{q}"] = pltpu.SemaphoreType.DMA

    @functools.partial(
        pl.kernel,
        out_type=jax.ShapeDtypeStruct((NC, npad, p), jnp.float32),
        mesh=mesh,
        compiler_params=pltpu.CompilerParams(use_tc_tiling_on_sc=False),
        scratch_types=scratch,
    )
    def edge_pass(table, srcf, dstf, zrows, out, **scr):
        sidx = [scr[f"sidx{b}"] for b in range(NB)]
        didx = [scr[f"didx{b}"] for b in range(NB)]
        rows = [scr[f"rows{b}"] for b in range(NB)]
        gsem = [scr[f"gsem{b}"] for b in range(NB)]
        ssem = [scr[f"ssem{b}"] for b in range(NB)]
        isem = [[scr[f"isem{b}_{q}"] for q in range(2)] for b in range(NB)]
        acc = scr["acc"]

        c = lax.axis_index("c")
        s = lax.axis_index("s")
        wid = s * NC + c
        rt0 = s * rt
        ebase = wid * t_edges
        pltpu.sync_copy(zrows.at[pl.ds(rt0, rt)], acc.at[pl.ds(rt0, rt)])
        plsc.subcore_barrier()

        def fire_idx(b, q, g):
            off = ebase + g * G
            pltpu.async_copy(srcf.at[pl.ds(off, G)], sidx[b].at[q],
                             isem[b][q])
            pltpu.async_copy(dstf.at[pl.ds(off, G)], didx[b].at[q],
                             isem[b][q])

        def wait_idx(b, q):
            cp = pltpu.make_async_copy(srcf.at[pl.ds(0, G)], sidx[b].at[q],
                                       isem[b][q])
            cp.wait()
            cp.wait()

        def fire_gather(b, q):
            pltpu.async_copy(table.at[sidx[b].at[q]], rows[b], gsem[b])

        def wait_gather(b, q):
            pltpu.make_async_copy(table.at[sidx[b].at[q]], rows[b],
                                  gsem[b]).wait()

        def fire_scatter(b, q):
            pltpu.async_copy(rows[b], acc.at[didx[b].at[q]], ssem[b],
                             add=True)

        def wait_scatter(b, q):
            pltpu.make_async_copy(rows[b], acc.at[didx[b].at[q]],
                                  ssem[b]).wait()

        def emit_iter(t_static, t_val):
            q = t_static % 2
            for b in range(NB):
                if t_static >= 1:
                    wait_scatter(b, 1 - q)
                if 1 <= t_static <= t_total - 2:
                    fire_idx(b, 1 - q, (t_val + 1) * NB + b)
                wait_idx(b, q)
                fire_gather(b, q)
                if not (t_static == 0 and b < 2):
                    p2 = (b - 2) % NB
                    q2 = q if b >= 2 else 1 - q
                    wait_gather(p2, q2)
                    fire_scatter(p2, q2)

        for g in range(2 * NB):
            fire_idx(g % NB, g // NB, g)
        emit_iter(0, 0)
        emit_iter(1, 1)

        def outer(t2, carry):
            t = 2 * t2 + 2
            emit_iter(2, t)
            emit_iter(3, t + 1)
            return carry

        lax.fori_loop(0, (t_total - 4) // 2, outer, 0)
        emit_iter(t_total - 2, t_total - 2)
        emit_iter(t_total - 1, t_total - 1)
        qf = (t_total - 1) % 2
        for b in (NB - 2, NB - 1):
            wait_gather(b, qf)
            fire_scatter(b, qf)
        for b in range(NB):
            wait_scatter(b, qf)

        plsc.subcore_barrier()
        pltpu.sync_copy(acc.at[pl.ds(rt0, rt)], out.at[c, pl.ds(rt0, rt)])

    return edge_pass


def kernel(x1, x2, edges, W1, b1, W2, b2, W3, b3, D1W, D1b, D2W, D2b):
    n = x1.shape[0]
    e = edges.shape[1]
    p = 8
    npad = n + 96
    t_edges = -(-e // (NW * G * NB * 2)) * G * NB * 2
    et = NW * t_edges
    pad_e = et - e

    pad_idx = n + (jnp.arange(pad_e, dtype=jnp.int32) % 32)
    srcf = jnp.concatenate([edges[0], pad_idx])
    dstf = jnp.concatenate([edges[1], pad_idx])
    zrows = jnp.zeros((npad, p), jnp.float32)

    edge_pass = _make_edge_pass(npad, p, t_edges)

    def pad_table(u):
        return jnp.zeros((npad, p), jnp.float32).at[:n, :u.shape[1]].set(u)

    ones_t = zrows.at[:, 0].set(1.0)
    cnt = edge_pass(ones_t, dstf, dstf, zrows)
    deg = 1.0 + cnt[0, :n, 0] + cnt[1, :n, 0]
    dis = lax.rsqrt(deg)[:, None]

    u1 = dis * (x1 @ W1)
    s1 = edge_pass(pad_table(u1), srcf, dstf, zrows)
    h1 = jax.nn.relu(dis * (s1[0, :n, :5] + s1[1, :n, :5] + u1) + b1)
    u2 = dis * (h1 @ W2)
    s2 = edge_pass(pad_table(u2), srcf, dstf, zrows)
    h2 = jax.nn.relu(dis * (s2[0, :n, :5] + s2[1, :n, :5] + u2) + b2)
    u3 = dis * (h2 @ W3)
    s3 = edge_pass(pad_table(u3), srcf, dstf, zrows)
    h3 = dis * (s3[0, :n, :1] + s3[1, :n, :1] + u3) + b3

    m = h3.mean()
    v = jnp.concatenate([m[None], x2])
    v = jax.nn.relu(v @ D1W + D1b)
    out = (v @ D2W + D2b)[0]
    return jnp.tanh(out)


# speedup vs baseline: 76.2856x; 1.4374x over previous
import functools

import jax
import jax.numpy as jnp
from jax import lax
from jax.experimental import pallas as pl
from jax.experimental.pallas import tpu as pltpu
from jax.experimental.pallas import tpu_sc as plsc

NC = 2
NS = 16
NW = NC * NS
G = 1024
NB = 5


def _make_edge_pass(npad, p, t_edges):
    rt = npad // NS
    t_total = t_edges // (G * NB)
    assert t_total >= 4 and t_total % 2 == 0
    mesh = plsc.VectorSubcoreMesh(core_axis_name="c", subcore_axis_name="s")

    scratch = dict(acc=pltpu.VMEM_SHARED((npad, p), jnp.float32))
    for b in range(NB):
        scratch[f"sidx{b}"] = pltpu.VMEM((2, G), jnp.int32)
        scratch[f"didx{b}"] = pltpu.VMEM((2, G), jnp.int32)
        scratch[f"rows{b}"] = pltpu.VMEM((G, p), jnp.float32)
        scratch[f"gsem{b}"] = pltpu.SemaphoreType.DMA
        scratch[f"ssem{b}"] = pltpu.SemaphoreType.DMA
        for q in range(2):
            scratch[f"isem{b}_{q}"] = pltpu.SemaphoreType.DMA

    @functools.partial(
        pl.kernel,
        out_type=jax.ShapeDtypeStruct((NC, npad, p), jnp.float32),
        mesh=mesh,
        compiler_params=pltpu.CompilerParams(use_tc_tiling_on_sc=False),
        scratch_types=scratch,
    )
    def edge_pass(table, srcf, dstf, zrows, out, **scr):
        sidx = [scr[f"sidx{b}"] for b in range(NB)]
        didx = [scr[f"didx{b}"] for b in range(NB)]
        rows = [scr[f"rows{b}"] for b in range(NB)]
        gsem = [scr[f"gsem{b}"] for b in range(NB)]
        ssem = [scr[f"ssem{b}"] for b in range(NB)]
        isem = [[scr[f"isem{b}_{q}"] for q in range(2)] for b in range(NB)]
        acc = scr["acc"]

        c = lax.axis_index("c")
        s = lax.axis_index("s")
        wid = s * NC + c
        rt0 = s * rt
        pltpu.sync_copy(zrows.at[pl.ds(rt0, rt)], acc.at[pl.ds(rt0, rt)])
        plsc.subcore_barrier()

        def fire_idx(b, q, g):
            off = (g * NW + wid) * G
            pltpu.async_copy(srcf.at[pl.ds(off, G)], sidx[b].at[q],
                             isem[b][q])
            pltpu.async_copy(dstf.at[pl.ds(off, G)], didx[b].at[q],
                             isem[b][q])

        def wait_idx(b, q):
            cp = pltpu.make_async_copy(srcf.at[pl.ds(0, G)], sidx[b].at[q],
                                       isem[b][q])
            cp.wait()
            cp.wait()

        def fire_gather(b, q):
            pltpu.async_copy(table.at[sidx[b].at[q]], rows[b], gsem[b])

        def wait_gather(b, q):
            pltpu.make_async_copy(table.at[sidx[b].at[q]], rows[b],
                                  gsem[b]).wait()

        def fire_scatter(b, q):
            pltpu.async_copy(rows[b], acc.at[didx[b].at[q]], ssem[b],
                             add=True)

        def wait_scatter(b, q):
            pltpu.make_async_copy(rows[b], acc.at[didx[b].at[q]],
                                  ssem[b]).wait()

        def emit_iter(t_static, t_val):
            q = t_static % 2
            for b in range(NB):
                if t_static >= 1:
                    wait_scatter(b, 1 - q)
                if 1 <= t_static <= t_total - 2:
                    fire_idx(b, 1 - q, (t_val + 1) * NB + b)
                wait_idx(b, q)
                fire_gather(b, q)
                if not (t_static == 0 and b < 2):
                    p2 = (b - 2) % NB
                    q2 = q if b >= 2 else 1 - q
                    wait_gather(p2, q2)
                    fire_scatter(p2, q2)

        for g in range(2 * NB):
            fire_idx(g % NB, g // NB, g)
        emit_iter(0, 0)
        emit_iter(1, 1)

        def outer(t2, carry):
            t = 2 * t2 + 2
            emit_iter(2, t)
            emit_iter(3, t + 1)
            return carry

        lax.fori_loop(0, (t_total - 4) // 2, outer, 0)
        emit_iter(t_total - 2, t_total - 2)
        emit_iter(t_total - 1, t_total - 1)
        qf = (t_total - 1) % 2
        for b in (NB - 2, NB - 1):
            wait_gather(b, qf)
            fire_scatter(b, qf)
        for b in range(NB):
            wait_scatter(b, qf)

        plsc.subcore_barrier()
        pltpu.sync_copy(acc.at[pl.ds(rt0, rt)], out.at[c, pl.ds(rt0, rt)])

    return edge_pass


def kernel(x1, x2, edges, W1, b1, W2, b2, W3, b3, D1W, D1b, D2W, D2b):
    n = x1.shape[0]
    e = edges.shape[1]
    p = 8
    npad = n + 2400
    t_edges = -(-e // (NW * G * NB * 2)) * G * NB * 2
    et = NW * t_edges
    pad_e = et - e

    pad_idx = n + (jnp.arange(pad_e, dtype=jnp.int32) % 2048)
    srcf = jnp.concatenate([edges[0], pad_idx])
    dstf = jnp.concatenate([edges[1], pad_idx])
    zrows = jnp.zeros((npad, p), jnp.float32)

    edge_pass = _make_edge_pass(npad, p, t_edges)

    def pad_table(u):
        return jnp.zeros((npad, p), jnp.float32).at[:n, :u.shape[1]].set(u)

    ones_t = zrows.at[:, 0].set(1.0)
    cnt = edge_pass(ones_t, dstf, dstf, zrows)
    deg = 1.0 + cnt[0, :n, 0] + cnt[1, :n, 0]
    dis = lax.rsqrt(deg)[:, None]

    u1 = dis * (x1 @ W1)
    s1 = edge_pass(pad_table(u1), srcf, dstf, zrows)
    h1 = jax.nn.relu(dis * (s1[0, :n, :5] + s1[1, :n, :5] + u1) + b1)
    u2 = dis * (h1 @ W2)
    s2 = edge_pass(pad_table(u2), srcf, dstf, zrows)
    h2 = jax.nn.relu(dis * (s2[0, :n, :5] + s2[1, :n, :5] + u2) + b2)
    u3 = dis * (h2 @ W3)
    s3 = edge_pass(pad_table(u3), srcf, dstf, zrows)
    h3 = dis * (s3[0, :n, :1] + s3[1, :n, :1] + u3) + b3

    m = h3.mean()
    v = jnp.concatenate([m[None], x2])
    v = jax.nn.relu(v @ D1W + D1b)
    out = (v @ D2W + D2b)[0]
    return jnp.tanh(out)


# speedup vs baseline: 79.2651x; 1.0391x over previous
import functools

import jax
import jax.numpy as jnp
from jax import lax
from jax.experimental import pallas as pl
from jax.experimental.pallas import tpu as pltpu
from jax.experimental.pallas import tpu_sc as plsc

NC = 2
NS = 16
NW = NC * NS
G = 1024
NB = 5


def _make_edge_pass(npad, p, t_edges):
    rt = npad // NS
    t_total = t_edges // (G * NB)
    assert t_total >= 4 and t_total % 2 == 0
    mesh = plsc.VectorSubcoreMesh(core_axis_name="c", subcore_axis_name="s")

    scratch = dict(acc=pltpu.VMEM_SHARED((npad, p), jnp.float32))
    for b in range(NB):
        scratch[f"sidx{b}"] = pltpu.VMEM((2, G), jnp.int32)
        scratch[f"didx{b}"] = pltpu.VMEM((2, G), jnp.int32)
        scratch[f"rows{b}"] = pltpu.VMEM((G, p), jnp.float32)
        scratch[f"gsem{b}"] = pltpu.SemaphoreType.DMA
        scratch[f"ssem{b}"] = pltpu.SemaphoreType.DMA
        for q in range(2):
            scratch[f"isem{b}_{q}"] = pltpu.SemaphoreType.DMA

    @functools.partial(
        pl.kernel,
        out_type=jax.ShapeDtypeStruct((NC, npad, p), jnp.float32),
        mesh=mesh,
        compiler_params=pltpu.CompilerParams(use_tc_tiling_on_sc=False),
        scratch_types=scratch,
    )
    def edge_pass(table, srcf, dstf, zrows, out, **scr):
        sidx = [scr[f"sidx{b}"] for b in range(NB)]
        didx = [scr[f"didx{b}"] for b in range(NB)]
        rows = [scr[f"rows{b}"] for b in range(NB)]
        gsem = [scr[f"gsem{b}"] for b in range(NB)]
        ssem = [scr[f"ssem{b}"] for b in range(NB)]
        isem = [[scr[f"isem{b}_{q}"] for q in range(2)] for b in range(NB)]
        acc = scr["acc"]

        c = lax.axis_index("c")
        s = lax.axis_index("s")
        wid = s * NC + c
        rt0 = s * rt
        pltpu.sync_copy(zrows.at[pl.ds(rt0, rt)], acc.at[pl.ds(rt0, rt)])
        plsc.subcore_barrier()

        def fire_idx(b, q, g):
            off = (g * NW + wid) * G
            pltpu.async_copy(srcf.at[pl.ds(off, G)], sidx[b].at[q],
                             isem[b][q])
            pltpu.async_copy(dstf.at[pl.ds(off, G)], didx[b].at[q],
                             isem[b][q])

        def wait_idx(b, q):
            cp = pltpu.make_async_copy(srcf.at[pl.ds(0, G)], sidx[b].at[q],
                                       isem[b][q])
            cp.wait()
            cp.wait()

        def fire_gather(b, q):
            pltpu.async_copy(table.at[sidx[b].at[q]], rows[b], gsem[b])

        def wait_gather(b, q):
            pltpu.make_async_copy(table.at[sidx[b].at[q]], rows[b],
                                  gsem[b]).wait()

        def fire_scatter(b, q):
            pltpu.async_copy(rows[b], acc.at[didx[b].at[q]], ssem[b],
                             add=True)

        def wait_scatter(b, q):
            pltpu.make_async_copy(rows[b], acc.at[didx[b].at[q]],
                                  ssem[b]).wait()

        def emit_iter(t_static, t_val):
            q = t_static % 2
            for b in range(NB):
                if t_static >= 1:
                    wait_scatter(b, 1 - q)
                if 1 <= t_static <= t_total - 2:
                    fire_idx(b, 1 - q, (t_val + 1) * NB + b)
                wait_idx(b, q)
                fire_gather(b, q)
                if not (t_static == 0 and b < 2):
                    p2 = (b - 2) % NB
                    q2 = q if b >= 2 else 1 - q
                    wait_gather(p2, q2)
                    fire_scatter(p2, q2)

        for g in range(2 * NB):
            fire_idx(g % NB, g // NB, g)
        emit_iter(0, 0)
        emit_iter(1, 1)

        def outer(t2, carry):
            t = 2 * t2 + 2
            emit_iter(2, t)
            emit_iter(3, t + 1)
            return carry

        lax.fori_loop(0, (t_total - 4) // 2, outer, 0)
        emit_iter(t_total - 2, t_total - 2)
        emit_iter(t_total - 1, t_total - 1)
        qf = (t_total - 1) % 2
        for b in (NB - 2, NB - 1):
            wait_gather(b, qf)
            fire_scatter(b, qf)
        for b in range(NB):
            wait_scatter(b, qf)

        plsc.subcore_barrier()
        pltpu.sync_copy(acc.at[pl.ds(rt0, rt)], out.at[c, pl.ds(rt0, rt)])

    return edge_pass


def kernel(x1, x2, edges, W1, b1, W2, b2, W3, b3, D1W, D1b, D2W, D2b):
    n = x1.shape[0]
    e = edges.shape[1]
    p = 8
    npad = n + 2400
    t_edges = -(-e // (NW * G * NB * 2)) * G * NB * 2
    et = NW * t_edges
    pad_e = et - e

    pad_idx = n + (jnp.arange(pad_e, dtype=jnp.int32) % 2048)
    srcf = jnp.concatenate([edges[0], pad_idx])
    dstf = jnp.concatenate([edges[1], pad_idx])
    zrows = jnp.zeros((npad, p), jnp.float32)

    edge_pass = _make_edge_pass(npad, p, t_edges)

    def to_table(ut):
        return jnp.transpose(ut)

    def pad_w(w):
        return jnp.zeros((p, p), jnp.float32).at[:w.shape[0], :w.shape[1]].set(w)

    ones_t = zrows.at[:, 0].set(1.0)
    cnt = edge_pass(ones_t, dstf, dstf, zrows)
    cnt_t = jnp.transpose(cnt, (0, 2, 1))
    deg = 1.0 + cnt_t[0, 0] + cnt_t[1, 0]
    dis = lax.rsqrt(deg)[None, :]

    x1t = jnp.zeros((p, npad), jnp.float32).at[:3, :n].set(jnp.transpose(x1))

    def layer(w, b, ut, relu):
        s = edge_pass(to_table(ut), srcf, dstf, zrows)
        st = jnp.transpose(s, (0, 2, 1))
        h = dis * (st[0] + st[1] + ut) + jnp.pad(b, (0, p - b.shape[0]))[:, None]
        if relu:
            h = jax.nn.relu(h)
        return h

    u1 = dis * (pad_w(W1).T @ x1t)
    h1 = layer(W1, b1, u1, True)
    u2 = dis * (pad_w(W2).T @ h1)
    h2 = layer(W2, b2, u2, True)
    u3 = dis * (pad_w(W3).T @ h2)
    h3 = layer(W3, b3, u3, False)

    m = h3[0, :n].mean()
    v = jnp.concatenate([m[None], x2])
    v = jax.nn.relu(v @ D1W + D1b)
    out = (v @ D2W + D2b)[0]
    return jnp.tanh(out)


# speedup vs baseline: 167.5238x; 2.1135x over previous
import functools

import jax
import jax.numpy as jnp
from jax import lax
from jax.experimental import pallas as pl
from jax.experimental.pallas import tpu as pltpu
from jax.experimental.pallas import tpu_sc as plsc

NC = 2
NS = 16
NW = NC * NS
G = 1024
NB = 5
TW = 5
TCH = 1600


def _wid_slices(rt):
    c = lax.axis_index("c")
    s = lax.axis_index("s")
    wid = s * NC + c
    return c, s, wid, s * rt


def _edge_schedule(t_total, fire_idx, wait_idx, fire_gather, wait_gather,
                   fire_scatter, wait_scatter):
    def emit_iter(t_static, t_val):
        q = t_static % 2
        for b in range(NB):
            if t_static >= 1:
                wait_scatter(b, 1 - q)
            if 1 <= t_static <= t_total - 2:
                fire_idx(b, 1 - q, (t_val + 1) * NB + b)
            wait_idx(b, q)
            fire_gather(b, q)
            if not (t_static == 0 and b < 2):
                p2 = (b - 2) % NB
                q2 = q if b >= 2 else 1 - q
                wait_gather(p2, q2)
                fire_scatter(p2, q2)

    for g in range(2 * NB):
        fire_idx(g % NB, g // NB, g)
    emit_iter(0, 0)
    emit_iter(1, 1)

    def outer(t2, carry):
        t = 2 * t2 + 2
        emit_iter(2, t)
        emit_iter(3, t + 1)
        return carry

    lax.fori_loop(0, (t_total - 4) // 2, outer, 0)
    emit_iter(t_total - 2, t_total - 2)
    emit_iter(t_total - 1, t_total - 1)
    qf = (t_total - 1) % 2
    for b in (NB - 2, NB - 1):
        wait_gather(b, qf)
        fire_scatter(b, qf)
    for b in range(NB):
        wait_scatter(b, qf)


def _make_layer_pass(npad, t_edges):
    rt = npad // NS
    t_total = t_edges // (G * NB)
    assert t_total >= 4 and t_total % 2 == 0 and rt % TCH == 0
    mesh = plsc.VectorSubcoreMesh(core_axis_name="c", subcore_axis_name="s")

    scratch = dict(
        acc=pltpu.VMEM_SHARED((npad, TW), jnp.float32),
        tslab=pltpu.VMEM((TW, TCH), jnp.float32),
        tstage=pltpu.VMEM((TCH, TW), jnp.float32),
        tsem=pltpu.SemaphoreType.DMA,
        zsem=pltpu.SemaphoreType.DMA,
    )
    for b in range(NB):
        scratch[f"sidx{b}"] = pltpu.VMEM((2, G), jnp.int32)
        scratch[f"didx{b}"] = pltpu.VMEM((2, G), jnp.int32)
        scratch[f"rows{b}"] = pltpu.VMEM((G, TW), jnp.float32)
        scratch[f"gsem{b}"] = pltpu.SemaphoreType.DMA
        scratch[f"ssem{b}"] = pltpu.SemaphoreType.DMA
        for q in range(2):
            scratch[f"isem{b}_{q}"] = pltpu.SemaphoreType.DMA

    @functools.partial(
        pl.kernel,
        out_type=(jax.ShapeDtypeStruct((NC, TW, npad), jnp.float32),
                  jax.ShapeDtypeStruct((NC, npad, TW), jnp.float32)),
        mesh=mesh,
        compiler_params=pltpu.CompilerParams(use_tc_tiling_on_sc=False, needs_layout_passes=False),
        scratch_types=scratch,
    )
    def layer_pass(ut, srcf, dstf, zrows, out, tbl, **scr):
        sidx = [scr[f"sidx{b}"] for b in range(NB)]
        didx = [scr[f"didx{b}"] for b in range(NB)]
        rows = [scr[f"rows{b}"] for b in range(NB)]
        gsem = [scr[f"gsem{b}"] for b in range(NB)]
        ssem = [scr[f"ssem{b}"] for b in range(NB)]
        isem = [[scr[f"isem{b}_{q}"] for q in range(2)] for b in range(NB)]
        acc = scr["acc"]
        table = tbl.at[lax.axis_index("c")]
        tslab, tstage = scr["tslab"], scr["tstage"]
        tsem, zsem = scr["tsem"], scr["zsem"]

        c, s, wid, rt0 = _wid_slices(rt)
        iota = lax.iota(jnp.int32, 16)
        jful = [jnp.full((16,), j, jnp.int32) for j in range(TW)]

        for ch in range(rt // TCH):
            pltpu.async_copy(zrows, acc.at[pl.ds(rt0 + ch * TCH, TCH)], zsem)
        for ch in range(rt // TCH):
            n0 = rt0 + ch * TCH
            cp = pltpu.async_copy(ut.at[:, pl.ds(n0, TCH)], tslab, tsem)
            cp.wait()

            def trn(i, carry):
                base = i * 16
                for j in range(TW):
                    v = tslab[j, pl.ds(base, 16)]
                    plsc.store_scatter(tstage, [base + iota, jful[j]], v)
                return carry

            lax.fori_loop(0, TCH // 16, trn, 0)
            pltpu.async_copy(tstage, table.at[pl.ds(n0, TCH)], tsem).wait()
        for ch in range(rt // TCH):
            pltpu.make_async_copy(zrows, acc.at[pl.ds(rt0 + ch * TCH, TCH)],
                                  zsem).wait()
        plsc.subcore_barrier()

        def fire_idx(b, q, g):
            off = (g * NW + wid) * G
            pltpu.async_copy(srcf.at[pl.ds(off, G)], sidx[b].at[q],
                             isem[b][q])
            pltpu.async_copy(dstf.at[pl.ds(off, G)], didx[b].at[q],
                             isem[b][q])

        def wait_idx(b, q):
            cp = pltpu.make_async_copy(srcf.at[pl.ds(0, G)], sidx[b].at[q],
                                       isem[b][q])
            cp.wait()
            cp.wait()

        def fire_gather(b, q):
            pltpu.async_copy(table.at[sidx[b].at[q]], rows[b], gsem[b])

        def wait_gather(b, q):
            pltpu.make_async_copy(table.at[sidx[b].at[q]], rows[b],
                                  gsem[b]).wait()

        def fire_scatter(b, q):
            pltpu.async_copy(rows[b], acc.at[didx[b].at[q]], ssem[b],
                             add=True)

        def wait_scatter(b, q):
            pltpu.make_async_copy(rows[b], acc.at[didx[b].at[q]],
                                  ssem[b]).wait()

        _edge_schedule(t_total, fire_idx, wait_idx, fire_gather, wait_gather,
                       fire_scatter, wait_scatter)
        plsc.subcore_barrier()

        for ch in range(rt // TCH):
            n0 = rt0 + ch * TCH
            pltpu.async_copy(acc.at[pl.ds(n0, TCH)], tstage, tsem).wait()

            def trn_out(i, carry):
                base = i * 16
                for j in range(TW):
                    v = plsc.load_gather(tstage, [base + iota, jful[j]])
                    tslab[j, pl.ds(base, 16)] = v
                return carry

            lax.fori_loop(0, TCH // 16, trn_out, 0)
            pltpu.async_copy(tslab, out.at[c, :, pl.ds(n0, TCH)],
                             tsem).wait()

    return layer_pass


def _make_deg_pass(npad, t_edges):
    rt = npad // NS
    t_total = t_edges // (G * NB)
    mesh = plsc.VectorSubcoreMesh(core_axis_name="c", subcore_axis_name="s")

    scratch = dict(
        acc=pltpu.VMEM_SHARED((npad,), jnp.float32),
        ones=pltpu.VMEM((G,), jnp.float32),
        osem=pltpu.SemaphoreType.DMA,
    )
    for b in range(NB):
        scratch[f"didx{b}"] = pltpu.VMEM((2, G), jnp.int32)
        scratch[f"ssem{b}"] = pltpu.SemaphoreType.DMA
        for q in range(2):
            scratch[f"isem{b}_{q}"] = pltpu.SemaphoreType.DMA

    @functools.partial(
        pl.kernel,
        out_type=jax.ShapeDtypeStruct((NC, npad), jnp.float32),
        mesh=mesh,
        compiler_params=pltpu.CompilerParams(use_tc_tiling_on_sc=False, needs_layout_passes=False),
        scratch_types=scratch,
    )
    def deg_pass(ones_in, dstf, zeros1, out, **scr):
        didx = [scr[f"didx{b}"] for b in range(NB)]
        ssem = [scr[f"ssem{b}"] for b in range(NB)]
        isem = [[scr[f"isem{b}_{q}"] for q in range(2)] for b in range(NB)]
        acc, ones, osem = scr["acc"], scr["ones"], scr["osem"]

        c, s, wid, rt0 = _wid_slices(rt)
        pltpu.async_copy(ones_in, ones, osem)
        pltpu.async_copy(zeros1, acc.at[pl.ds(rt0, rt)], osem)
        pltpu.make_async_copy(ones_in, ones, osem).wait()
        pltpu.make_async_copy(zeros1, acc.at[pl.ds(rt0, rt)], osem).wait()
        plsc.subcore_barrier()

        def fire_idx(b, q, g):
            off = (g * NW + wid) * G
            pltpu.async_copy(dstf.at[pl.ds(off, G)], didx[b].at[q],
                             isem[b][q])

        def wait_idx(b, q):
            pltpu.make_async_copy(dstf.at[pl.ds(0, G)], didx[b].at[q],
                                  isem[b][q]).wait()

        def fire_scatter(b, q):
            pltpu.async_copy(ones, acc.at[didx[b].at[q]], ssem[b], add=True)

        def wait_scatter(b, q):
            pltpu.make_async_copy(ones, acc.at[didx[b].at[q]],
                                  ssem[b]).wait()

        def noop(b, q):
            pass

        _edge_schedule(t_total, fire_idx, wait_idx, noop, noop,
                       fire_scatter, wait_scatter)
        plsc.subcore_barrier()
        pltpu.sync_copy(acc.at[pl.ds(rt0, rt)], out.at[c, pl.ds(rt0, rt)])

    return deg_pass


def kernel(x1, x2, edges, W1, b1, W2, b2, W3, b3, D1W, D1b, D2W, D2b):
    n = x1.shape[0]
    e = edges.shape[1]
    npad = n + 2400
    t_edges = -(-e // (NW * G * NB * 2)) * G * NB * 2
    pad_e = NW * t_edges - e

    pad_idx = n + (jnp.arange(pad_e, dtype=jnp.int32) % 2048)
    srcf = jnp.concatenate([edges[0], pad_idx])
    dstf = jnp.concatenate([edges[1], pad_idx])
    zrows = jnp.zeros((TCH, TW), jnp.float32)

    layer_pass = _make_layer_pass(npad, t_edges)
    deg_pass = _make_deg_pass(npad, t_edges)

    def pad_w(w):
        return jnp.zeros((TW, TW), jnp.float32).at[:w.shape[0], :w.shape[1]].set(w)

    cnt = deg_pass(jnp.ones((G,), jnp.float32), dstf,
                   jnp.zeros((npad // NS,), jnp.float32))
    deg = 1.0 + cnt[0] + cnt[1]
    dis = lax.rsqrt(deg)[None, :]

    x1t = jnp.zeros((TW, npad), jnp.float32).at[:3, :n].set(jnp.transpose(x1))

    def layer(w, b, ut, relu):
        st, _ = layer_pass(ut, srcf, dstf, zrows)
        h = dis * (st[0] + st[1] + ut) + jnp.pad(b, (0, TW - b.shape[0]))[:, None]
        return jax.nn.relu(h) if relu else h

    u1 = dis * (pad_w(W1).T @ x1t)
    h1 = layer(W1, b1, u1, True)
    u2 = dis * (pad_w(W2).T @ h1)
    h2 = layer(W2, b2, u2, True)
    u3 = dis * (pad_w(W3).T @ h2)
    h3 = layer(W3, b3, u3, False)

    m = h3[0, :n].mean()
    v = jnp.concatenate([m[None], x2])
    v = jax.nn.relu(v @ D1W + D1b)
    out = (v @ D2W + D2b)[0]
    return jnp.tanh(out)
